# baseline (speedup 1.0000x reference)
import functools

import jax
import jax.numpy as jnp
from jax import lax
from jax.experimental import pallas as pl
from jax.experimental.pallas import tpu as pltpu
from jax.experimental.pallas import tpu_sc as plsc

NC, NS, L = 2, 16, 16
NW = NC * NS
B = 16384
D = 64
BPW = B // NW
CH = 128
NCH = BPW // CH
G = D // L
RPB = 16
NBLK = BPW // RPB

_mesh = plsc.VectorSubcoreMesh(core_axis_name="c", subcore_axis_name="s")


@functools.partial(
    pl.kernel,
    out_type=jax.ShapeDtypeStruct((B,), jnp.float32),
    mesh=_mesh,
    scratch_types=[
        pltpu.VMEM((NCH, CH), jnp.int32),
        pltpu.VMEM((NCH, CH), jnp.int32),
        pltpu.VMEM((NCH, CH), jnp.int32),
        pltpu.VMEM((BPW, D), jnp.float32),
        pltpu.VMEM((BPW, D), jnp.float32),
        pltpu.VMEM((BPW, D), jnp.float32),
        pltpu.VMEM((BPW,), jnp.float32),
        pltpu.SemaphoreType.DMA,
    ],
    compiler_params=pltpu.CompilerParams(use_tc_tiling_on_sc=False),
)
def _transe(head_h, rel_h, tail_h, ent_h, relemb_h, out_h,
            ih_v, ir_v, it_v, h_v, r_v, t_v, o_v, sem):
    wid = lax.axis_index("s") * NC + lax.axis_index("c")

    pltpu.sync_copy(head_h.at[wid], ih_v)
    pltpu.sync_copy(rel_h.at[wid], ir_v)
    pltpu.sync_copy(tail_h.at[wid], it_v)

    copies = []
    for c in range(NCH):
        rows = pl.ds(c * CH, CH)
        copies.append(pltpu.async_copy(ent_h.at[ih_v.at[c]], h_v.at[rows], sem))
        copies.append(pltpu.async_copy(relemb_h.at[ir_v.at[c]], r_v.at[rows], sem))
        copies.append(pltpu.async_copy(ent_h.at[it_v.at[c]], t_v.at[rows], sem))
    for cp in copies:
        cp.wait()

    iot = lax.iota(jnp.int32, L)
    _dnums = lax.GatherDimensionNumbers(
        offset_dims=(), collapsed_slice_dims=(0,), start_index_map=(0,))

    def _perm(v, idx):
        return lax.gather(v, idx.reshape(L, 1), _dnums, (1,),
                          mode=lax.GatherScatterMode.PROMISE_IN_BOUNDS)

    def _hsum(v):
        for s in (8, 4, 2, 1):
            v = v + _perm(v, jnp.bitwise_xor(iot, s))
        return v

    def blk_body(blk, carry):
        rbase = blk * RPB
        outv = jnp.zeros((L,), jnp.float32)
        for rr in range(RPB):
            row = rbase + rr
            acc = jnp.zeros((L,), jnp.float32)
            for g in range(G):
                sl = pl.ds(g * L, L)
                acc = acc + jnp.abs(h_v[row, sl] + r_v[row, sl] - t_v[row, sl])
            outv = jnp.where(iot == rr, _hsum(acc), outv)
        o_v[pl.ds(rbase, RPB)] = -outv
        return carry

    lax.fori_loop(0, NBLK, blk_body, 0)
    pltpu.sync_copy(o_v, out_h.at[pl.ds(wid * BPW, BPW)])


def kernel(head, relation, tail, entity_emb, relation_emb):
    head3 = head.astype(jnp.int32).reshape(NW, NCH, CH)
    rel3 = relation.astype(jnp.int32).reshape(NW, NCH, CH)
    tail3 = tail.astype(jnp.int32).reshape(NW, NCH, CH)
    return _transe(head3, rel3, tail3, entity_emb, relation_emb)


# speedup vs baseline: 1.6841x; 1.6841x over previous
import functools

import jax
import jax.numpy as jnp
from jax import lax
from jax.experimental import pallas as pl
from jax.experimental.pallas import tpu as pltpu
from jax.experimental.pallas import tpu_sc as plsc

NC, NS, L = 2, 16, 16
NW = NC * NS
B = 16384
D = 64
BPW = B // NW
G = D // L
RPB = 16
NBLK = BPW // RPB

_mesh = plsc.VectorSubcoreMesh(core_axis_name="c", subcore_axis_name="s")


@functools.partial(
    pl.kernel,
    out_type=jax.ShapeDtypeStruct((B,), jnp.float32),
    mesh=_mesh,
    scratch_types=[
        pltpu.VMEM((BPW,), jnp.int32),
        pltpu.VMEM((BPW,), jnp.int32),
        pltpu.VMEM((BPW,), jnp.int32),
        pltpu.VMEM((BPW // 2, 2 * D), jnp.float32),
        pltpu.VMEM((BPW // 2, 2 * D), jnp.float32),
        pltpu.VMEM((BPW // 2, 2 * D), jnp.float32),
        pltpu.VMEM((BPW,), jnp.float32),
        pltpu.SemaphoreType.DMA,
    ],
)
def _transe(head_h, rel_h, tail_h, ent_h, relemb_h, out_h,
            ih_v, ir_v, it_v, h_v, r_v, t_v, o_v, sem):
    wid = lax.axis_index("s") * NC + lax.axis_index("c")

    pltpu.sync_copy(head_h.at[wid], ih_v)
    pltpu.sync_copy(rel_h.at[wid], ir_v)
    pltpu.sync_copy(tail_h.at[wid], it_v)

    def fire(g, c):
        base = g * L
        ihv = ih_v[pl.ds(base, L)]
        irv = ir_v[pl.ds(base, L)]
        itv = it_v[pl.ds(base, L)]
        for rr in range(L):
            i = base + rr
            half = pl.ds((rr % 2) * D, D)
            pltpu.async_copy(ent_h.at[ihv[rr]], h_v.at[i // 2, half], sem)
            pltpu.async_copy(relemb_h.at[irv[rr]], r_v.at[i // 2, half], sem)
            pltpu.async_copy(ent_h.at[itv[rr]], t_v.at[i // 2, half], sem)
        return c

    lax.fori_loop(0, BPW // L, fire, 0)
    pltpu.make_async_copy(ent_h.at[pl.ds(0, BPW)], h_v, sem).wait()
    pltpu.make_async_copy(ent_h.at[pl.ds(0, BPW)], r_v, sem).wait()
    pltpu.make_async_copy(ent_h.at[pl.ds(0, BPW)], t_v, sem).wait()

    iot = lax.iota(jnp.int32, L)
    _dnums = lax.GatherDimensionNumbers(
        offset_dims=(), collapsed_slice_dims=(0,), start_index_map=(0,))

    def _perm(v, idx):
        return lax.gather(v, idx.reshape(L, 1), _dnums, (1,),
                          mode=lax.GatherScatterMode.PROMISE_IN_BOUNDS)

    def _hsum(v):
        for s in (8, 4, 2, 1):
            v = v + _perm(v, jnp.bitwise_xor(iot, s))
        return v

    def blk_body(blk, carry):
        rbase = blk * RPB
        outv = jnp.zeros((L,), jnp.float32)
        for rr in range(RPB):
            row = rbase + rr
            acc = jnp.zeros((L,), jnp.float32)
            for g in range(G):
                sl = pl.ds((rr % 2) * D + g * L, L)
                acc = acc + jnp.abs(h_v[row // 2, sl] + r_v[row // 2, sl]
                                    - t_v[row // 2, sl])
            outv = jnp.where(iot == rr, _hsum(acc), outv)
        o_v[pl.ds(rbase, RPB)] = -outv
        return carry

    lax.fori_loop(0, NBLK, blk_body, 0)
    pltpu.sync_copy(o_v, out_h.at[pl.ds(wid * BPW, BPW)])


def kernel(head, relation, tail, entity_emb, relation_emb):
    head2 = head.astype(jnp.int32).reshape(NW, BPW)
    rel2 = relation.astype(jnp.int32).reshape(NW, BPW)
    tail2 = tail.astype(jnp.int32).reshape(NW, BPW)
    return _transe(head2, rel2, tail2, entity_emb, relation_emb)
